# Initial kernel scaffold; baseline (speedup 1.0000x reference)
import jax
import jax.numpy as jnp
from jax.experimental import pallas as pl


def kernel(output, target):
    raise NotImplementedError("write your pallas kernel here")


# speedup vs baseline: 1.5304x; 1.5304x over previous
import functools

import jax
import jax.numpy as jnp
from jax.experimental import pallas as pl
from jax.experimental.pallas import tpu as pltpu

N_CLASSES = 19
EPS = 1e-08
IGNORE_INDEX = 0

_HB = 64


def _dice_kernel(out_ref, tgt_ref, dsc_ref, acc_ref):
    b = pl.program_id(0)
    h = pl.program_id(1)
    nb = pl.num_programs(0)
    nh = pl.num_programs(1)
    first = jnp.logical_and(b == 0, h == 0)
    last = jnp.logical_and(b == nb - 1, h == nh - 1)

    @pl.when(first)
    def _():
        acc_ref[...] = jnp.zeros_like(acc_ref)

    x = out_ref[0]
    lt = tgt_ref[0]

    best_val = x[0]
    best_idx = jnp.zeros_like(lt)
    for c in range(1, N_CLASSES):
        cur = x[c]
        gt = cur > best_val
        best_idx = jnp.where(gt, jnp.int32(c), best_idx)
        best_val = jnp.where(gt, cur, best_val)

    valid = jnp.logical_and(lt != IGNORE_INDEX,
                            jnp.logical_and(lt >= 0, lt < N_CLASSES))
    cls = jax.lax.broadcasted_iota(jnp.int32, (N_CLASSES,) + lt.shape, 0)
    a_oh = jnp.where(jnp.logical_and(cls == lt[None], valid[None]),
                     jnp.float32(1.0), jnp.float32(0.0))
    b_oh = jnp.where(cls == best_idx[None], jnp.float32(1.0), jnp.float32(0.0))
    n = lt.shape[0] * lt.shape[1]
    a2 = a_oh.reshape(N_CLASSES, n)
    b2 = b_oh.reshape(N_CLASSES, n)
    acc_ref[...] += jax.lax.dot_general(
        a2, b2, (((1,), (1,)), ((), ())), preferred_element_type=jnp.float32)

    @pl.when(last)
    def _():
        mat = acc_ref[...]
        i0 = jax.lax.broadcasted_iota(jnp.int32, (N_CLASSES, N_CLASSES), 0)
        i1 = jax.lax.broadcasted_iota(jnp.int32, (N_CLASSES, N_CLASSES), 1)
        eye = jnp.where(i0 == i1, jnp.float32(1.0), jnp.float32(0.0))
        ones = jnp.ones((N_CLASSES, 1), jnp.float32)
        tp = jnp.dot(mat * eye, ones)
        fp = jnp.dot(mat, ones)
        fn = jax.lax.dot_general(mat, ones, (((0,), (0,)), ((), ())))
        precision = tp / (fp + EPS)
        recall = tp / (fn + EPS)
        dsc = 2.0 * precision * recall / (precision + recall + EPS)
        gt_empty = (tp + fn) == 0
        pred_empty = (tp + fp) == 0
        nan = jnp.float32(jnp.nan)
        dsc = jnp.where(jnp.logical_and(gt_empty, pred_empty), nan, dsc)
        dsc = jnp.where(jnp.logical_and(gt_empty, ~pred_empty),
                        jnp.float32(0.0), dsc)
        row = jax.lax.broadcasted_iota(jnp.int32, (N_CLASSES, 1), 0)
        dsc = jnp.where(row == IGNORE_INDEX, nan, dsc)
        dsc_ref[...] = dsc


@jax.jit
def kernel(output, target):
    bsz, nc, hh, ww = output.shape
    target = target.astype(jnp.int32)
    grid = (bsz, hh // _HB)
    dsc = pl.pallas_call(
        _dice_kernel,
        grid=grid,
        in_specs=[
            pl.BlockSpec((1, nc, _HB, ww), lambda b, h: (b, 0, h, 0)),
            pl.BlockSpec((1, _HB, ww), lambda b, h: (b, h, 0)),
        ],
        out_specs=pl.BlockSpec((N_CLASSES, 1), lambda b, h: (0, 0)),
        out_shape=jax.ShapeDtypeStruct((N_CLASSES, 1), jnp.float32),
        scratch_shapes=[pltpu.VMEM((N_CLASSES, N_CLASSES), jnp.float32)],
    )(output, target)
    return dsc.reshape(N_CLASSES)
